# Initial kernel scaffold; baseline (speedup 1.0000x reference)
import jax
import jax.numpy as jnp
from jax.experimental import pallas as pl


def kernel(x, edge_index, num_timeline, W1, b1, W2, b2, Wc1, bc1, Wc2, bc2):
    raise NotImplementedError("write your pallas kernel here")


# speedup vs baseline: 1.2567x; 1.2567x over previous
import jax
import jax.numpy as jnp
from jax.experimental import pallas as pl
from jax.experimental.pallas import tpu as pltpu

N = 10000
E = 320000
D_IN = 128
D_H = 256
D_OUT = 10


def _gcn_conv(x, src, dst, W, b):
    deg = jnp.zeros((x.shape[0],), dtype=x.dtype).at[dst].add(1.0) + 1.0
    dinv = jax.lax.rsqrt(deg)
    h = x @ W
    coeff = (dinv[src] * dinv[dst])[:, None]
    out = jnp.zeros((x.shape[0], W.shape[1]), dtype=x.dtype).at[dst].add(h[src] * coeff)
    out = out + h * (dinv * dinv)[:, None]
    return out + b


def _classifier_body(hrow_ref, Wc1_ref, bc1_ref, Wc2_ref, bc2_ref, out_ref):
    z = jnp.maximum(hrow_ref[...] @ Wc1_ref[...] + bc1_ref[...][None, :], 0.0)
    o = z @ Wc2_ref[...] + bc2_ref[...][None, :]
    out_ref[...] = 1.0 / (1.0 + jnp.exp(-o))


def kernel(x, edge_index, num_timeline, W1, b1, W2, b2, Wc1, bc1, Wc2, bc2):
    src = edge_index[0]
    dst = edge_index[1]
    h = jax.nn.relu(_gcn_conv(x, src, dst, W1, b1))
    h = _gcn_conv(h, src, dst, W2, b2)
    idx = num_timeline[0] - 1
    hrow = jax.lax.dynamic_slice_in_dim(h, idx, 1, axis=0)
    Wc2p = jnp.zeros((D_H, 128), Wc2.dtype).at[:, :D_OUT].set(Wc2)
    bc2p = jnp.zeros((128,), bc2.dtype).at[:D_OUT].set(bc2)
    out = pl.pallas_call(
        _classifier_body,
        out_shape=jax.ShapeDtypeStruct((1, 128), jnp.float32),
    )(hrow, Wc1, bc1, Wc2p, bc2p)
    return out[:, :D_OUT]


# speedup vs baseline: 42.6467x; 33.9363x over previous
import jax
import jax.numpy as jnp
from jax import lax
from jax.experimental import pallas as pl
from jax.experimental.pallas import tpu as pltpu
from jax.experimental.pallas import tpu_sc as plsc

N = 10000
E = 320000
D_IN = 128
D_H = 256
D_OUT = 10

NT = 16
EPT = E // NT
NPAD = 10240
SLICE = NPAD // NT
WIN = 2000
NWIN = EPT // WIN
CSEG = 2048
CCAP = WIN + 16
GB = 16
CAPR = 4096
DUMMY = CAPR


def _newton_rsqrt(d):
    i = plsc.bitcast(d, jnp.int32)
    y = plsc.bitcast(jnp.int32(0x5F3759DF) - (i >> 1), jnp.float32)
    for _ in range(3):
        y = y * (1.5 - 0.5 * d * y * y)
    return y


def _sload(ref, i):
    return ref[pl.ds(i, 16)][0]


def _sstore(ref, i, val):
    lanes = lax.iota(jnp.int32, 16)
    ivec = jnp.broadcast_to(i, (16,)).astype(jnp.int32)
    vvec = jnp.broadcast_to(val, (16,))
    plsc.store_scatter(ref, [ivec], vvec, mask=lanes == 0)


def _sc_body(
    x_hbm, src_hbm, dst_hbm, ntb_hbm,
    W1_hbm, b1_hbm, W2_hbm, b2_hbm, Wc1_hbm, bc1_hbm, Wc2p_hbm, bc2p_hbm,
    out_hbm,
    srcw, dstw, dinvl, flagl, Ul, wvl,
    matchb, srcm, pm, xrows, W1l, b1l, upart, pbuf, acc,
    cntl, cntall, kidxl, idxgbuf, idxsbuf, sclbuf, ntl,
    part_sh, degdinv_sh, C_sh, W1_sh, cnts_sh, flag_sh, U_sh, wv_sh, kidx_sh,
    a_sh, uparts_sh,
    sem,
):
    wid = lax.axis_index("s")
    i32 = jnp.int32
    f32 = jnp.float32
    lanes = lax.iota(i32, 16)

    base_e = wid * EPT
    pltpu.sync_copy(ntb_hbm, ntl)
    idx = _sload(ntl, 0) - 1

    zf = jnp.zeros((16,), f32)
    ones = jnp.ones((16,), f32)

    def zero_hist(k, _):
        dinvl[pl.ds(k * 16, 16)] = zf
        return 0

    lax.fori_loop(0, NPAD // 16, zero_hist, 0)

    def hist_win(w, _):
        pltpu.sync_copy(dst_hbm.at[pl.ds(base_e + w * WIN, WIN)],
                        dstw.at[pl.ds(0, WIN)])

        def hist(k, _):
            iv = dstw[pl.ds(k * 16, 16)]
            plsc.addupdate_scatter(dinvl, [iv], ones)
            return 0

        lax.fori_loop(0, WIN // 16, hist, 0)
        return 0

    lax.fori_loop(0, NWIN, hist_win, 0)

    pltpu.sync_copy(dinvl, part_sh.at[wid])
    plsc.subcore_barrier()
    sbase = wid * SLICE

    def accum_zero(k, _):
        acc[pl.ds(k * 16, 16)] = zf
        return 0

    lax.fori_loop(0, SLICE // 16, accum_zero, 0)

    def accum_tile(t, _):
        pltpu.sync_copy(part_sh.at[t, pl.ds(sbase, SLICE)], pbuf.at[pl.ds(0, SLICE)])

        def add_v(k, _):
            acc[pl.ds(k * 16, 16)] = acc[pl.ds(k * 16, 16)] + pbuf[pl.ds(k * 16, 16)]
            return 0

        lax.fori_loop(0, SLICE // 16, add_v, 0)
        return 0

    lax.fori_loop(0, NT, accum_tile, 0)

    def dinv_v(k, _):
        d = acc[pl.ds(k * 16, 16)] + 1.0
        acc[pl.ds(k * 16, 16)] = _newton_rsqrt(d)
        return 0

    lax.fori_loop(0, SLICE // 16, dinv_v, 0)
    pltpu.sync_copy(acc.at[pl.ds(0, SLICE)], degdinv_sh.at[pl.ds(sbase, SLICE)])
    plsc.subcore_barrier()
    pltpu.sync_copy(degdinv_sh, dinvl)

    def scan_b_win(w, cnt):
        pltpu.sync_copy(dst_hbm.at[pl.ds(base_e + w * WIN, WIN)],
                        dstw.at[pl.ds(0, WIN)])
        pltpu.sync_copy(src_hbm.at[pl.ds(base_e + w * WIN, WIN)],
                        srcw.at[pl.ds(0, WIN)])

        def scan_b_vec(v, cnt):
            off = v * 16
            dv = dstw[pl.ds(off, 16)]
            nhit = jnp.sum(jnp.where(dv == idx, 1, 0))

            def slow(cnt):
                def lane(l, cnt):
                    d = _sload(dstw, off + l)
                    _sstore(matchb, jnp.minimum(cnt, CSEG + 16),
                            _sload(srcw, off + l))
                    return cnt + jnp.where(d == idx, 1, 0)

                return lax.fori_loop(0, 16, lane, cnt)

            return lax.cond(nhit > 0, slow, lambda c: c, cnt)

        return lax.fori_loop(0, WIN // 16, scan_b_vec, cnt)

    cnt_b = lax.fori_loop(0, NWIN, scan_b_win, jnp.int32(0))

    def flush(ch, _):
        pltpu.sync_copy(matchb.at[pl.ds(ch * 256, 256)],
                        C_sh.at[wid, pl.ds(ch * 256, 256)])
        return 0

    lax.fori_loop(0, (jnp.minimum(cnt_b, CSEG) + 255) // 256, flush, 0)
    cntl[...] = jnp.broadcast_to(cnt_b, (16,)).astype(i32)
    pltpu.sync_copy(cntl, cnts_sh.at[wid])
    plsc.subcore_barrier()

    @pl.when(wid == 0)
    def _dedup():
        mone = jnp.full((16,), -1, i32)

        def zero_flag(k, _):
            flagl[pl.ds(k * 16, 16)] = mone
            return 0

        lax.fori_loop(0, NPAD // 16, zero_flag, 0)

        def load_cnts(t, _):
            pltpu.sync_copy(cnts_sh.at[t], cntall.at[pl.ds(t * 16, 16)])
            return 0

        lax.fori_loop(0, NT, load_cnts, 0)

        def dedup_one(s, K, extra):
            f = _sload(flagl, s)
            isnew = f < 0
            slot = jnp.where(isnew, K, f)
            _sstore(flagl, s, slot)
            _sstore(Ul, slot, s)
            dv = _sload(dinvl, s) + extra
            old = jnp.where(isnew, 0.0, _sload(wvl, slot))
            _sstore(wvl, slot, old + dv)
            return K + jnp.where(isnew, 1, 0)

        def seg(t, K):
            cnt = _sload(cntall, t * 16)

            def from_c(K):
                def load(ch, _):
                    pltpu.sync_copy(C_sh.at[t, pl.ds(ch * 256, 256)],
                                    matchb.at[pl.ds(ch * 256, 256)])
                    return 0

                lax.fori_loop(0, (cnt + 255) // 256, load, 0)

                def ent(i, K):
                    return dedup_one(_sload(matchb, i), K, jnp.float32(0.0))

                return lax.fori_loop(0, cnt, ent, K)

            def rescan(K):
                def rw(w, K):
                    tb = t * EPT + w * WIN
                    pltpu.sync_copy(dst_hbm.at[pl.ds(tb, WIN)],
                                    dstw.at[pl.ds(0, WIN)])
                    pltpu.sync_copy(src_hbm.at[pl.ds(tb, WIN)],
                                    srcw.at[pl.ds(0, WIN)])

                    def rv(v, K):
                        off = v * 16
                        dv = dstw[pl.ds(off, 16)]
                        nhit = jnp.sum(jnp.where(dv == idx, 1, 0))

                        def slow(K):
                            def lane(l, K):
                                d = _sload(dstw, off + l)

                                def hitfn(K):
                                    return dedup_one(
                                        _sload(srcw, off + l), K,
                                        jnp.float32(0.0))

                                return lax.cond(d == idx, hitfn,
                                                lambda K: K, K)

                            return lax.fori_loop(0, 16, lane, K)

                        return lax.cond(nhit > 0, slow, lambda K: K, K)

                    return lax.fori_loop(0, WIN // 16, rv, K)

                return lax.fori_loop(0, NWIN, rw, K)

            return lax.cond(cnt <= CSEG, from_c, rescan, K)

        K = lax.fori_loop(0, NT, seg, jnp.int32(0))
        K = dedup_one(idx, K, jnp.float32(0.0))
        kidxl[...] = jnp.broadcast_to(K, (16,)).astype(i32)
        pltpu.sync_copy(kidxl, kidx_sh)
        pltpu.sync_copy(flagl, flag_sh)
        pltpu.sync_copy(Ul, U_sh)
        pltpu.sync_copy(wvl, wv_sh)

    @pl.when(wid == 1)
    def _stage_w1():
        pltpu.sync_copy(W1_hbm, W1_sh)

    plsc.subcore_barrier()
    pltpu.sync_copy(flag_sh, flagl)
    pltpu.sync_copy(U_sh, Ul)
    pltpu.sync_copy(wv_sh, wvl)
    pltpu.sync_copy(kidx_sh, kidxl)
    K = _sload(kidxl, 0)

    pltpu.sync_copy(b1_hbm, b1l)

    def zero_upart(k, _):
        upart[pl.ds(k * 16, 16)] = zf
        return 0

    lax.fori_loop(0, D_H // 16, zero_upart, 0)

    def zero_xrows(r, _):
        for c in range(D_IN // 16):
            xrows[r, pl.ds(c * 16, 16)] = zf
        return 0

    lax.fori_loop(0, GB, zero_xrows, 0)

    def run_batch(ivec, svec, scl):
        idxgbuf[...] = ivec
        idxsbuf[...] = svec
        sclbuf[pl.ds(0, 16)] = scl
        pltpu.async_copy(x_hbm.at[idxgbuf], xrows, sem).wait()

        def scale_row(r, _):
            s = _sload(sclbuf, r)
            sv = jnp.broadcast_to(s, (16,))
            for c in range(D_IN // 16):
                xrows[r, pl.ds(c * 16, 16)] = xrows[r, pl.ds(c * 16, 16)] * sv
            return 0

        lax.fori_loop(0, GB, scale_row, 0)
        pltpu.sync_copy(xrows, a_sh.at[idxsbuf], add=True)

    npass = (K + CAPR - 1) // CAPR

    def one_pass(p, _):
        lo = p * CAPR
        nloc = jnp.minimum(CAPR, K - lo)
        nblk = (nloc + GB - 1) // GB
        nmine = jnp.maximum(0, (nblk - wid + NT - 1) // NT)

        def rezero_xrows(r, _):
            for c in range(D_IN // 16):
                xrows[r, pl.ds(c * 16, 16)] = zf
            return 0

        lax.fori_loop(0, GB, rezero_xrows, 0)

        def zero_blk(ii, _):
            b = wid + ii * NT
            pltpu.sync_copy(xrows, a_sh.at[pl.ds(b * GB, GB)])
            return 0

        lax.fori_loop(0, nmine, zero_blk, 0)
        plsc.subcore_barrier()

        def scan_d_win(w, _):
            pltpu.sync_copy(dst_hbm.at[pl.ds(base_e + w * WIN, WIN)],
                            dstw.at[pl.ds(0, WIN)])
            pltpu.sync_copy(src_hbm.at[pl.ds(base_e + w * WIN, WIN)],
                            srcw.at[pl.ds(0, WIN)])

            def scan_d_vec(v, cnt):
                off = v * 16
                dv = dstw[pl.ds(off, 16)]
                fv = plsc.load_gather(flagl, [dv])
                hit = (fv >= lo) & (fv < lo + nloc)
                nhit = jnp.sum(jnp.where(hit, 1, 0))

                def slow(cnt):
                    def lane(l, cnt):
                        q = _sload(flagl, _sload(dstw, off + l))
                        _sstore(srcm, cnt, _sload(srcw, off + l))
                        _sstore(pm, cnt, q - lo)
                        take = (q >= lo) & (q < lo + nloc)
                        return cnt + jnp.where(take, 1, 0)

                    return lax.fori_loop(0, 16, lane, cnt)

                return lax.cond(nhit > 0, slow, lambda c: c, cnt)

            cnt = lax.fori_loop(0, WIN // 16, scan_d_vec, jnp.int32(0))

            def batch(b, _):
                valid = b * GB + lanes < cnt
                sv = srcm[pl.ds(b * GB, GB)]
                pv = pm[pl.ds(b * GB, GB)]
                ivec = jnp.where(valid, sv, 0)
                svec = jnp.where(valid, pv, DUMMY)
                dg = plsc.load_gather(dinvl, [ivec])
                scl = jnp.where(valid, dg, 0.0)
                run_batch(ivec, svec, scl)
                return 0

            lax.fori_loop(0, (cnt + GB - 1) // GB, batch, 0)
            return 0

        lax.fori_loop(0, NWIN, scan_d_win, 0)

        def self_blk(ii, _):
            b = wid + ii * NT
            uv = Ul[pl.ds(lo + b * GB, GB)]
            valid = b * GB + lanes < nloc
            ivec = jnp.where(valid, uv, 0)
            svec = jnp.where(valid, b * GB + lanes, DUMMY)
            dg = plsc.load_gather(dinvl, [ivec])
            scl = jnp.where(valid, dg, 0.0)
            run_batch(ivec, svec, scl)
            return 0

        lax.fori_loop(0, nmine, self_blk, 0)
        plsc.subcore_barrier()

        def slot_blk(ii, _):
            b = wid + ii * NT

            def one_slot(jj, _):
                jl = b * GB + jj
                jg = lo + jl

                def work(_):
                    pltpu.sync_copy(a_sh.at[jl], pbuf.at[pl.ds(0, D_IN)])
                    duj = _sload(dinvl, _sload(Ul, jg))
                    wj = _sload(wvl, jg)

                    accs = tuple(
                        jnp.zeros((16,), f32) for _ in range(D_H // 16))
                    for h in range(2):
                        pltpu.sync_copy(W1_sh.at[pl.ds(h * 64, 64)], W1l)

                        def mv(k, accs, h=h):
                            s = _sload(pbuf, h * 64 + k)
                            sv = jnp.broadcast_to(s, (16,))
                            return tuple(
                                accs[c] + sv * W1l[k, pl.ds(c * 16, 16)]
                                for c in range(D_H // 16)
                            )

                        accs = lax.fori_loop(0, 64, mv, accs)
                    for c in range(D_H // 16):
                        h1 = jnp.maximum(
                            b1l[pl.ds(c * 16, 16)] + duj * accs[c], 0.0)
                        upart[pl.ds(c * 16, 16)] = (
                            upart[pl.ds(c * 16, 16)] + wj * h1)
                    return 0

                return lax.cond(jl < nloc, work, lambda _: 0, 0)

            lax.fori_loop(0, GB, one_slot, 0)
            return 0

        lax.fori_loop(0, nmine, slot_blk, 0)
        plsc.subcore_barrier()
        return 0

    lax.fori_loop(0, npass, one_pass, 0)
    pltpu.sync_copy(upart, uparts_sh.at[wid])
    plsc.subcore_barrier()

    @pl.when(wid == 0)
    def _tail():
        def zero_u(k, _):
            acc[pl.ds(k * 16, 16)] = zf
            return 0

        lax.fori_loop(0, D_H // 16, zero_u, 0)

        def sum_parts(t, _):
            pltpu.sync_copy(uparts_sh.at[t], upart)

            def add_v(k, _):
                acc[pl.ds(k * 16, 16)] = (
                    acc[pl.ds(k * 16, 16)] + upart[pl.ds(k * 16, 16)])
                return 0

            lax.fori_loop(0, D_H // 16, add_v, 0)
            return 0

        lax.fori_loop(0, NT, sum_parts, 0)
        dii = _sload(dinvl, idx)

        def matvec256(src_ref, w_hbm, b_hbm, scale, relu):
            pltpu.sync_copy(b_hbm, b1l)
            accs = tuple(jnp.zeros((16,), f32) for _ in range(D_H // 16))
            for h in range(4):
                pltpu.sync_copy(w_hbm.at[pl.ds(h * 64, 64)], W1l)

                def mv(k, accs, h=h):
                    s = _sload(src_ref, h * 64 + k)
                    sv = jnp.broadcast_to(s, (16,))
                    return tuple(
                        accs[c] + sv * W1l[k, pl.ds(c * 16, 16)]
                        for c in range(D_H // 16)
                    )

                accs = lax.fori_loop(0, 64, mv, accs)
            out = []
            for c in range(D_H // 16):
                v = b1l[pl.ds(c * 16, 16)] + scale * accs[c]
                out.append(jnp.maximum(v, 0.0) if relu else v)
            return out

        h2 = matvec256(acc, W2_hbm, b2_hbm, dii, relu=False)
        for c in range(D_H // 16):
            pbuf[pl.ds(c * 16, 16)] = h2[c]
        z = matvec256(pbuf, Wc1_hbm, bc1_hbm, jnp.float32(1.0), relu=True)
        for c in range(D_H // 16):
            acc[pl.ds(c * 16, 16)] = z[c]
        pltpu.sync_copy(Wc2p_hbm, W1l.at[pl.ds(0, 16)])
        pltpu.sync_copy(bc2p_hbm, sclbuf.at[pl.ds(0, 16)])

        def mvout(k, o):
            s = _sload(acc, k)
            sv = jnp.broadcast_to(s, (16,))
            return o + sv * W1l[k // 16, pl.ds((k % 16) * 16, 16)]

        o = lax.fori_loop(0, D_H, mvout, jnp.zeros((16,), f32))
        o = o + sclbuf[pl.ds(0, 16)]
        res = 1.0 / (1.0 + jnp.exp(-o))
        upart[pl.ds(0, 16)] = res
        pltpu.sync_copy(upart.at[pl.ds(0, 16)], out_hbm)


@jax.jit
def _sc_call(x, srcv, dstv, ntb, W1, b1, W2, b2, Wc1, bc1, Wc2p, bc2p):
    mesh = plsc.VectorSubcoreMesh(
        core_axis_name="c", subcore_axis_name="s", num_cores=1)
    f32 = jnp.float32
    i32 = jnp.int32
    kern = pl.kernel(
        _sc_body,
        out_type=jax.ShapeDtypeStruct((16,), f32),
        mesh=mesh,
        compiler_params=pltpu.CompilerParams(needs_layout_passes=False),
        scratch_types=[
            pltpu.VMEM((WIN + 16,), i32),
            pltpu.VMEM((WIN + 16,), i32),
            pltpu.VMEM((NPAD,), f32),
            pltpu.VMEM((NPAD,), i32),
            pltpu.VMEM((NPAD,), i32),
            pltpu.VMEM((NPAD,), f32),
            pltpu.VMEM((CSEG + 48,), i32),
            pltpu.VMEM((CCAP,), i32),
            pltpu.VMEM((CCAP,), i32),
            pltpu.VMEM((GB, D_IN), f32),
            pltpu.VMEM((64, D_H), f32),
            pltpu.VMEM((D_H,), f32),
            pltpu.VMEM((D_H,), f32),
            pltpu.VMEM((SLICE + 16,), f32),
            pltpu.VMEM((SLICE + 16,), f32),
            pltpu.VMEM((16,), i32),
            pltpu.VMEM((NT * 16 + 16,), i32),
            pltpu.VMEM((16,), i32),
            pltpu.VMEM((GB,), i32),
            pltpu.VMEM((GB,), i32),
            pltpu.VMEM((32,), f32),
            pltpu.VMEM((16,), i32),
            pltpu.VMEM_SHARED((NT, NPAD), f32),
            pltpu.VMEM_SHARED((NPAD,), f32),
            pltpu.VMEM_SHARED((NT, CSEG), i32),
            pltpu.VMEM_SHARED((128, D_H), f32),
            pltpu.VMEM_SHARED((NT, 16), i32),
            pltpu.VMEM_SHARED((NPAD,), i32),
            pltpu.VMEM_SHARED((NPAD,), i32),
            pltpu.VMEM_SHARED((NPAD,), f32),
            pltpu.VMEM_SHARED((16,), i32),
            pltpu.VMEM_SHARED((CAPR + 16, D_IN), f32),
            pltpu.VMEM_SHARED((NT, D_H), f32),
            pltpu.SemaphoreType.DMA,
        ],
    )
    return kern(x, srcv, dstv, ntb, W1, b1, W2, b2, Wc1, bc1, Wc2p, bc2p)


def kernel(x, edge_index, num_timeline, W1, b1, W2, b2, Wc1, bc1, Wc2, bc2):
    srcv = edge_index[0]
    dstv = edge_index[1]
    ntb = jnp.broadcast_to(num_timeline, (16,)).astype(jnp.int32)
    Wc2p = jnp.zeros((D_H, 16), Wc2.dtype).at[:, :D_OUT].set(Wc2)
    Wc2p = Wc2p.reshape(16, D_H)
    bc2p = jnp.zeros((16,), bc2.dtype).at[:D_OUT].set(bc2)
    out = _sc_call(x, srcv, dstv, ntb, W1, b1, W2, b2, Wc1, bc1, Wc2p, bc2p)
    return out[:D_OUT][None, :]


# speedup vs baseline: 50.8165x; 1.1916x over previous
import jax
import jax.numpy as jnp
from jax import lax
from jax.experimental import pallas as pl
from jax.experimental.pallas import tpu as pltpu
from jax.experimental.pallas import tpu_sc as plsc

N = 10000
E = 320000
D_IN = 128
D_H = 256
D_OUT = 10

NT = 16
EPT = E // NT
NPAD = 10240
SLICE = NPAD // NT
WIN = 2000
NWIN = EPT // WIN
CSEG = 2048
CCAP = WIN + 16
GB = 16
CAPR = 4096
DUMMY = CAPR
TCB = 512
NBLK = NPAD // TCB


def _newton_rsqrt(d):
    i = plsc.bitcast(d, jnp.int32)
    y = plsc.bitcast(jnp.int32(0x5F3759DF) - (i >> 1), jnp.float32)
    for _ in range(3):
        y = y * (1.5 - 0.5 * d * y * y)
    return y


def _sload(ref, i):
    return ref[pl.ds(i, 16)][0]


def _sstore(ref, i, val):
    lanes = lax.iota(jnp.int32, 16)
    ivec = jnp.broadcast_to(i, (16,)).astype(jnp.int32)
    vvec = jnp.broadcast_to(val, (16,))
    plsc.store_scatter(ref, [ivec], vvec, mask=lanes == 0)


def _sc_body(
    x_hbm, src_hbm, dst_hbm, ntb_hbm,
    A_hbm, wv_hbm, scl_hbm, meta_hbm,
    srcw, dstw, dinvl, flagl, Ul, wvl,
    matchb, srcm, pm, xrows, pbuf, acc,
    cntl, cntall, kidxl, idxgbuf, idxsbuf, sclbuf, ntl,
    part_sh, degdinv_sh, C_sh, cnts_sh, flag_sh, U_sh, wv_sh, kidx_sh, a_sh,
    sem,
):
    wid = lax.axis_index("s")
    i32 = jnp.int32
    f32 = jnp.float32
    lanes = lax.iota(i32, 16)

    base_e = wid * EPT
    pltpu.sync_copy(ntb_hbm, ntl)
    idx = _sload(ntl, 0) - 1

    zf = jnp.zeros((16,), f32)
    ones = jnp.ones((16,), f32)

    def zero_hist(k, _):
        dinvl[pl.ds(k * 16, 16)] = zf
        return 0

    lax.fori_loop(0, NPAD // 16, zero_hist, 0)

    def hist_win(w, _):
        pltpu.sync_copy(dst_hbm.at[pl.ds(base_e + w * WIN, WIN)],
                        dstw.at[pl.ds(0, WIN)])

        def hist(k, _):
            iv = dstw[pl.ds(k * 16, 16)]
            plsc.addupdate_scatter(dinvl, [iv], ones)
            return 0

        lax.fori_loop(0, WIN // 16, hist, 0)
        return 0

    lax.fori_loop(0, NWIN, hist_win, 0)

    pltpu.sync_copy(dinvl, part_sh.at[wid])
    plsc.subcore_barrier()
    sbase = wid * SLICE

    def accum_zero(k, _):
        acc[pl.ds(k * 16, 16)] = zf
        return 0

    lax.fori_loop(0, SLICE // 16, accum_zero, 0)

    def accum_tile(t, _):
        pltpu.sync_copy(part_sh.at[t, pl.ds(sbase, SLICE)], pbuf.at[pl.ds(0, SLICE)])

        def add_v(k, _):
            acc[pl.ds(k * 16, 16)] = acc[pl.ds(k * 16, 16)] + pbuf[pl.ds(k * 16, 16)]
            return 0

        lax.fori_loop(0, SLICE // 16, add_v, 0)
        return 0

    lax.fori_loop(0, NT, accum_tile, 0)

    def dinv_v(k, _):
        d = acc[pl.ds(k * 16, 16)] + 1.0
        acc[pl.ds(k * 16, 16)] = _newton_rsqrt(d)
        return 0

    lax.fori_loop(0, SLICE // 16, dinv_v, 0)
    pltpu.sync_copy(acc.at[pl.ds(0, SLICE)], degdinv_sh.at[pl.ds(sbase, SLICE)])
    plsc.subcore_barrier()
    pltpu.sync_copy(degdinv_sh, dinvl)

    def scan_b_win(w, cnt):
        pltpu.sync_copy(dst_hbm.at[pl.ds(base_e + w * WIN, WIN)],
                        dstw.at[pl.ds(0, WIN)])
        pltpu.sync_copy(src_hbm.at[pl.ds(base_e + w * WIN, WIN)],
                        srcw.at[pl.ds(0, WIN)])

        def scan_b_vec(v, cnt):
            off = v * 16
            dv = dstw[pl.ds(off, 16)]
            nhit = jnp.sum(jnp.where(dv == idx, 1, 0))

            def slow(cnt):
                def lane(l, cnt):
                    d = _sload(dstw, off + l)
                    _sstore(matchb, jnp.minimum(cnt, CSEG + 16),
                            _sload(srcw, off + l))
                    return cnt + jnp.where(d == idx, 1, 0)

                return lax.fori_loop(0, 16, lane, cnt)

            return lax.cond(nhit > 0, slow, lambda c: c, cnt)

        return lax.fori_loop(0, WIN // 16, scan_b_vec, cnt)

    cnt_b = lax.fori_loop(0, NWIN, scan_b_win, jnp.int32(0))

    def flush(ch, _):
        pltpu.sync_copy(matchb.at[pl.ds(ch * 256, 256)],
                        C_sh.at[wid, pl.ds(ch * 256, 256)])
        return 0

    lax.fori_loop(0, (jnp.minimum(cnt_b, CSEG) + 255) // 256, flush, 0)
    cntl[...] = jnp.broadcast_to(cnt_b, (16,)).astype(i32)
    pltpu.sync_copy(cntl, cnts_sh.at[wid])
    plsc.subcore_barrier()

    @pl.when(wid == 0)
    def _dedup():
        mone = jnp.full((16,), -1, i32)

        def zero_flag(k, _):
            flagl[pl.ds(k * 16, 16)] = mone
            return 0

        lax.fori_loop(0, NPAD // 16, zero_flag, 0)

        def load_cnts(t, _):
            pltpu.sync_copy(cnts_sh.at[t], cntall.at[pl.ds(t * 16, 16)])
            return 0

        lax.fori_loop(0, NT, load_cnts, 0)

        def dedup_one(s, K, extra):
            f = _sload(flagl, s)
            isnew = f < 0
            slot = jnp.where(isnew, K, f)
            _sstore(flagl, s, slot)
            _sstore(Ul, slot, s)
            dv = _sload(dinvl, s) + extra
            old = jnp.where(isnew, 0.0, _sload(wvl, slot))
            _sstore(wvl, slot, old + dv)
            return K + jnp.where(isnew, 1, 0)

        def seg(t, K):
            cnt = _sload(cntall, t * 16)

            def from_c(K):
                def load(ch, _):
                    pltpu.sync_copy(C_sh.at[t, pl.ds(ch * 256, 256)],
                                    matchb.at[pl.ds(ch * 256, 256)])
                    return 0

                lax.fori_loop(0, (cnt + 255) // 256, load, 0)

                def ent(i, K):
                    return dedup_one(_sload(matchb, i), K, jnp.float32(0.0))

                return lax.fori_loop(0, cnt, ent, K)

            def rescan(K):
                def rw(w, K):
                    tb = t * EPT + w * WIN
                    pltpu.sync_copy(dst_hbm.at[pl.ds(tb, WIN)],
                                    dstw.at[pl.ds(0, WIN)])
                    pltpu.sync_copy(src_hbm.at[pl.ds(tb, WIN)],
                                    srcw.at[pl.ds(0, WIN)])

                    def rv(v, K):
                        off = v * 16
                        dv = dstw[pl.ds(off, 16)]
                        nhit = jnp.sum(jnp.where(dv == idx, 1, 0))

                        def slow(K):
                            def lane(l, K):
                                d = _sload(dstw, off + l)

                                def hitfn(K):
                                    return dedup_one(
                                        _sload(srcw, off + l), K,
                                        jnp.float32(0.0))

                                return lax.cond(d == idx, hitfn,
                                                lambda K: K, K)

                            return lax.fori_loop(0, 16, lane, K)

                        return lax.cond(nhit > 0, slow, lambda K: K, K)

                    return lax.fori_loop(0, WIN // 16, rv, K)

                return lax.fori_loop(0, NWIN, rw, K)

            return lax.cond(cnt <= CSEG, from_c, rescan, K)

        K = lax.fori_loop(0, NT, seg, jnp.int32(0))
        K = dedup_one(idx, K, jnp.float32(0.0))
        kidxl[...] = jnp.broadcast_to(K, (16,)).astype(i32)
        pltpu.sync_copy(kidxl, kidx_sh)
        pltpu.sync_copy(flagl, flag_sh)
        pltpu.sync_copy(Ul, U_sh)
        pltpu.sync_copy(wvl, wv_sh)

    plsc.subcore_barrier()
    pltpu.sync_copy(flag_sh, flagl)
    pltpu.sync_copy(U_sh, Ul)
    pltpu.sync_copy(wv_sh, wvl)
    pltpu.sync_copy(kidx_sh, kidxl)
    K = _sload(kidxl, 0)

    def run_batch(ivec, svec, scl):
        idxgbuf[...] = ivec
        idxsbuf[...] = svec
        sclbuf[pl.ds(0, 16)] = scl
        pltpu.async_copy(x_hbm.at[idxgbuf], xrows, sem).wait()

        def scale_row(r, _):
            s = _sload(sclbuf, r)
            sv = jnp.broadcast_to(s, (16,))
            for c in range(D_IN // 16):
                xrows[r, pl.ds(c * 16, 16)] = xrows[r, pl.ds(c * 16, 16)] * sv
            return 0

        lax.fori_loop(0, GB, scale_row, 0)
        pltpu.sync_copy(xrows, a_sh.at[idxsbuf], add=True)

    npass = (K + CAPR - 1) // CAPR
    zf = jnp.zeros((16,), jnp.float32)

    def one_pass(p, _):
        lo = p * CAPR
        nloc = jnp.minimum(CAPR, K - lo)
        nblk = (nloc + GB - 1) // GB
        nmine = jnp.maximum(0, (nblk - wid + NT - 1) // NT)

        def rezero_xrows(r, _):
            for c in range(D_IN // 16):
                xrows[r, pl.ds(c * 16, 16)] = zf
            return 0

        lax.fori_loop(0, GB, rezero_xrows, 0)

        def zero_blk(ii, _):
            b = wid + ii * NT
            pltpu.sync_copy(xrows, a_sh.at[pl.ds(b * GB, GB)])
            return 0

        lax.fori_loop(0, nmine, zero_blk, 0)
        plsc.subcore_barrier()

        def scan_d_win(w, _):
            pltpu.sync_copy(dst_hbm.at[pl.ds(base_e + w * WIN, WIN)],
                            dstw.at[pl.ds(0, WIN)])
            pltpu.sync_copy(src_hbm.at[pl.ds(base_e + w * WIN, WIN)],
                            srcw.at[pl.ds(0, WIN)])

            def scan_d_vec(v, cnt):
                off = v * 16
                dv = dstw[pl.ds(off, 16)]
                fv = plsc.load_gather(flagl, [dv])
                hit = (fv >= lo) & (fv < lo + nloc)
                nhit = jnp.sum(jnp.where(hit, 1, 0))

                def slow(cnt):
                    def lane(l, cnt):
                        q = _sload(flagl, _sload(dstw, off + l))
                        _sstore(srcm, cnt, _sload(srcw, off + l))
                        _sstore(pm, cnt, q - lo)
                        take = (q >= lo) & (q < lo + nloc)
                        return cnt + jnp.where(take, 1, 0)

                    return lax.fori_loop(0, 16, lane, cnt)

                return lax.cond(nhit > 0, slow, lambda c: c, cnt)

            cnt = lax.fori_loop(0, WIN // 16, scan_d_vec, jnp.int32(0))

            def batch(b, _):
                valid = b * GB + lax.iota(jnp.int32, 16) < cnt
                sv = srcm[pl.ds(b * GB, GB)]
                pv = pm[pl.ds(b * GB, GB)]
                ivec = jnp.where(valid, sv, 0)
                svec = jnp.where(valid, pv, DUMMY)
                dg = plsc.load_gather(dinvl, [ivec])
                scl = jnp.where(valid, dg, 0.0)
                run_batch(ivec, svec, scl)
                return 0

            lax.fori_loop(0, (cnt + GB - 1) // GB, batch, 0)
            return 0

        lax.fori_loop(0, NWIN, scan_d_win, 0)

        def self_blk(ii, _):
            b = wid + ii * NT
            uv = Ul[pl.ds(lo + b * GB, GB)]
            valid = b * GB + lax.iota(jnp.int32, 16) < nloc
            ivec = jnp.where(valid, uv, 0)
            svec = jnp.where(valid, b * GB + lax.iota(jnp.int32, 16), DUMMY)
            dg = plsc.load_gather(dinvl, [ivec])
            scl = jnp.where(valid, dg, 0.0)
            run_batch(ivec, svec, scl)
            return 0

        lax.fori_loop(0, nmine, self_blk, 0)
        plsc.subcore_barrier()

        def out_blk(ii, _):
            b = wid + ii * NT
            pltpu.sync_copy(a_sh.at[pl.ds(b * GB, GB)],
                            A_hbm.at[pl.ds(lo + b * GB, GB)])
            return 0

        lax.fori_loop(0, nmine, out_blk, 0)
        plsc.subcore_barrier()
        return 0

    lax.fori_loop(0, npass, one_pass, 0)

    pltpu.sync_copy(wvl.at[pl.ds(sbase, SLICE)], wv_hbm.at[pl.ds(sbase, SLICE)])

    def scl_v(k, _):
        off = sbase + k * 16
        uv = Ul[pl.ds(off, 16)]
        valid = off + lanes < K
        uv = jnp.where(valid, uv, 0)
        dg = plsc.load_gather(dinvl, [uv])
        acc[pl.ds(k * 16, 16)] = jnp.where(valid, dg, 0.0)
        return 0

    lax.fori_loop(0, SLICE // 16, scl_v, 0)
    pltpu.sync_copy(acc.at[pl.ds(0, SLICE)], scl_hbm.at[pl.ds(sbase, SLICE)])

    @pl.when(wid == 0)
    def _meta():
        dii = _sload(dinvl, idx)
        kf = K.astype(f32)
        mv = jnp.where(lanes == 0, kf, jnp.where(lanes == 1, dii, 0.0))
        sclbuf[pl.ds(0, 16)] = mv
        pltpu.sync_copy(sclbuf.at[pl.ds(0, 16)], meta_hbm)


@jax.jit
def _sc_call(x, srcv, dstv, ntb):
    mesh = plsc.VectorSubcoreMesh(
        core_axis_name="c", subcore_axis_name="s", num_cores=1)
    f32 = jnp.float32
    i32 = jnp.int32
    kern = pl.kernel(
        _sc_body,
        out_type=[
            jax.ShapeDtypeStruct((NPAD, D_IN), f32),
            jax.ShapeDtypeStruct((NPAD,), f32),
            jax.ShapeDtypeStruct((NPAD,), f32),
            jax.ShapeDtypeStruct((16,), f32),
        ],
        mesh=mesh,
        compiler_params=pltpu.CompilerParams(needs_layout_passes=False),
        scratch_types=[
            pltpu.VMEM((WIN + 16,), i32),
            pltpu.VMEM((WIN + 16,), i32),
            pltpu.VMEM((NPAD,), f32),
            pltpu.VMEM((NPAD,), i32),
            pltpu.VMEM((NPAD,), i32),
            pltpu.VMEM((NPAD,), f32),
            pltpu.VMEM((CSEG + 48,), i32),
            pltpu.VMEM((CCAP,), i32),
            pltpu.VMEM((CCAP,), i32),
            pltpu.VMEM((GB, D_IN), f32),
            pltpu.VMEM((SLICE + 16,), f32),
            pltpu.VMEM((SLICE + 16,), f32),
            pltpu.VMEM((16,), i32),
            pltpu.VMEM((NT * 16 + 16,), i32),
            pltpu.VMEM((16,), i32),
            pltpu.VMEM((GB,), i32),
            pltpu.VMEM((GB,), i32),
            pltpu.VMEM((32,), f32),
            pltpu.VMEM((16,), i32),
            pltpu.VMEM_SHARED((NT, NPAD), f32),
            pltpu.VMEM_SHARED((NPAD,), f32),
            pltpu.VMEM_SHARED((NT, CSEG), i32),
            pltpu.VMEM_SHARED((NT, 16), i32),
            pltpu.VMEM_SHARED((NPAD,), i32),
            pltpu.VMEM_SHARED((NPAD,), i32),
            pltpu.VMEM_SHARED((NPAD,), f32),
            pltpu.VMEM_SHARED((16,), i32),
            pltpu.VMEM_SHARED((CAPR + 16, D_IN), f32),
            pltpu.SemaphoreType.DMA,
        ],
    )
    return kern(x, srcv, dstv, ntb)


def _tc_body(A_ref, wv_ref, scl_ref, meta_ref,
             W1_ref, b1_ref, W2_ref, b2_ref,
             Wc1_ref, bc1_ref, Wc2_ref, bc2_ref,
             out_ref, u_scr):
    i = pl.program_id(0)
    f32 = jnp.float32
    Ki = meta_ref[0, 0].astype(jnp.int32)

    blk = A_ref[...]
    M = jnp.dot(blk, W1_ref[...], preferred_element_type=f32)
    h1 = jnp.maximum(b1_ref[...] + scl_ref[...] * M, 0.0)
    rid = i * TCB + lax.broadcasted_iota(jnp.int32, (TCB, 1), 0)
    mask = rid < Ki
    part = jnp.sum(jnp.where(mask, wv_ref[...] * h1, 0.0),
                   axis=0, keepdims=True)

    @pl.when(i == 0)
    def _init():
        u_scr[0:1, :] = part

    @pl.when(i > 0)
    def _acc():
        u_scr[0:1, :] = u_scr[0:1, :] + part

    @pl.when(i == NBLK - 1)
    def _tail():
        dii = meta_ref[0, 1]
        u = u_scr[0:1, :]
        h2 = b2_ref[...] + dii * jnp.dot(u, W2_ref[...],
                                         preferred_element_type=f32)
        z = jnp.maximum(jnp.dot(h2, Wc1_ref[...],
                                preferred_element_type=f32) + bc1_ref[...],
                        0.0)
        o = jnp.dot(z, Wc2_ref[...], preferred_element_type=f32) + bc2_ref[...]
        res = 1.0 / (1.0 + jnp.exp(-o))
        out_ref[...] = jnp.broadcast_to(res, (8, 128))


@jax.jit
def _tc_call(A, wv2, scl2, meta2, W1, b12, W2, b22, Wc1, bc12, Wc2p, bc2p):
    f32 = jnp.float32
    return pl.pallas_call(
        _tc_body,
        grid=(NBLK,),
        in_specs=[
            pl.BlockSpec((TCB, D_IN), lambda i: (i, 0)),
            pl.BlockSpec((TCB, 1), lambda i: (i, 0)),
            pl.BlockSpec((TCB, 1), lambda i: (i, 0)),
            pl.BlockSpec((1, 16), lambda i: (0, 0)),
            pl.BlockSpec((D_IN, D_H), lambda i: (0, 0)),
            pl.BlockSpec((1, D_H), lambda i: (0, 0)),
            pl.BlockSpec((D_H, D_H), lambda i: (0, 0)),
            pl.BlockSpec((1, D_H), lambda i: (0, 0)),
            pl.BlockSpec((D_H, D_H), lambda i: (0, 0)),
            pl.BlockSpec((1, D_H), lambda i: (0, 0)),
            pl.BlockSpec((D_H, 128), lambda i: (0, 0)),
            pl.BlockSpec((1, 128), lambda i: (0, 0)),
        ],
        out_specs=pl.BlockSpec((8, 128), lambda i: (0, 0)),
        out_shape=jax.ShapeDtypeStruct((8, 128), f32),
        scratch_shapes=[pltpu.VMEM((8, D_H), f32)],
    )(A, wv2, scl2, meta2, W1, b12, W2, b22, Wc1, bc12, Wc2p, bc2p)


def kernel(x, edge_index, num_timeline, W1, b1, W2, b2, Wc1, bc1, Wc2, bc2):
    srcv = edge_index[0]
    dstv = edge_index[1]
    ntb = jnp.broadcast_to(num_timeline, (16,)).astype(jnp.int32)
    A, wv, scl, meta = _sc_call(x, srcv, dstv, ntb)
    Wc2p = jnp.zeros((D_H, 128), Wc2.dtype).at[:, :D_OUT].set(Wc2)
    bc2p = jnp.zeros((1, 128), bc2.dtype).at[0, :D_OUT].set(bc2)
    out = _tc_call(A, wv[:, None], scl[:, None], meta[None, :],
                   W1, b1[None, :], W2, b2[None, :],
                   Wc1, bc1[None, :], Wc2p, bc2p)
    return out[0:1, :D_OUT]


# speedup vs baseline: 62.1704x; 1.2234x over previous
import jax
import jax.numpy as jnp
from jax import lax
from jax.experimental import pallas as pl
from jax.experimental.pallas import tpu as pltpu
from jax.experimental.pallas import tpu_sc as plsc

N = 10000
E = 320000
D_IN = 128
D_H = 256
D_OUT = 10

NT = 16
EPT = E // NT
NPAD = 10240
SLICE = NPAD // NT
WIN = 4000
NWIN = EPT // WIN
CSEG = 2048
CCAP = WIN + 48
GB = 16
CAPR = 4096
DUMMY = CAPR
TCB = 1024
NBLK = NPAD // TCB


def _newton_rsqrt(d):
    i = plsc.bitcast(d, jnp.int32)
    y = plsc.bitcast(jnp.int32(0x5F3759DF) - (i >> 1), jnp.float32)
    for _ in range(3):
        y = y * (1.5 - 0.5 * d * y * y)
    return y


def _sload(ref, i):
    return ref[pl.ds(i, 16)][0]


def _sstore(ref, i, val):
    lanes = lax.iota(jnp.int32, 16)
    ivec = jnp.broadcast_to(i, (16,)).astype(jnp.int32)
    vvec = jnp.broadcast_to(val, (16,))
    plsc.store_scatter(ref, [ivec], vvec, mask=lanes == 0)


def _sc_body(
    x_hbm, src_hbm, dst_hbm, ntb_hbm,
    A_hbm, wv_hbm, scl_hbm, meta_hbm,
    srcw, dstw, dinvl, flagl, Ul, wvl,
    matchb, srcm, pm, xrows, pbuf, acc,
    cntl, cntall, kidxl, idxgbuf, idxsbuf, sclbuf, ntl,
    part_sh, degdinv_sh, C_sh, cnts_sh, flag_sh, U_sh, kidx_sh, a_sh,
    sem,
):
    wid = lax.axis_index("s")
    i32 = jnp.int32
    f32 = jnp.float32
    lanes = lax.iota(i32, 16)

    base_e = wid * EPT
    pltpu.sync_copy(ntb_hbm, ntl)
    idx = _sload(ntl, 0) - 1

    zf = jnp.zeros((16,), f32)
    ones = jnp.ones((16,), f32)

    def zero_hist(k, _):
        dinvl[pl.ds(k * 16, 16)] = zf
        return 0

    lax.fori_loop(0, NPAD // 16, zero_hist, 0)

    def hist_win(w, cnt):
        pltpu.sync_copy(dst_hbm.at[pl.ds(base_e + w * WIN, WIN)],
                        dstw.at[pl.ds(0, WIN)])

        def hist(k, hit):
            iv0 = dstw[pl.ds(k * 32, 16)]
            iv1 = dstw[pl.ds(k * 32 + 16, 16)]
            plsc.addupdate_scatter(dinvl, [iv0], ones)
            plsc.addupdate_scatter(dinvl, [iv1], ones)
            return (hit + jnp.sum(jnp.where(iv0 == idx, 1, 0))
                    + jnp.sum(jnp.where(iv1 == idx, 1, 0)))

        hits = lax.fori_loop(0, WIN // 32, hist, jnp.int32(0))

        def with_src(cnt):
            pltpu.sync_copy(src_hbm.at[pl.ds(base_e + w * WIN, WIN)],
                            srcw.at[pl.ds(0, WIN)])

            def scan_b_vec(v, cnt):
                off = v * 16
                dv = dstw[pl.ds(off, 16)]
                nhit = jnp.sum(jnp.where(dv == idx, 1, 0))

                def slow(cnt):
                    def lane(l, cnt):
                        d = _sload(dstw, off + l)
                        _sstore(matchb, jnp.minimum(cnt, CSEG + 16),
                                _sload(srcw, off + l))
                        return cnt + jnp.where(d == idx, 1, 0)

                    return lax.fori_loop(0, 16, lane, cnt)

                return lax.cond(nhit > 0, slow, lambda c: c, cnt)

            return lax.fori_loop(0, WIN // 16, scan_b_vec, cnt)

        return lax.cond(hits > 0, with_src, lambda c: c, cnt)

    cnt_b = lax.fori_loop(0, NWIN, hist_win, jnp.int32(0))

    def flush(ch, _):
        pltpu.sync_copy(matchb.at[pl.ds(ch * 256, 256)],
                        C_sh.at[wid, pl.ds(ch * 256, 256)])
        return 0

    lax.fori_loop(0, (jnp.minimum(cnt_b, CSEG) + 255) // 256, flush, 0)
    cntl[...] = jnp.broadcast_to(cnt_b, (16,)).astype(i32)
    pltpu.sync_copy(cntl, cnts_sh.at[wid])

    pltpu.sync_copy(dinvl, part_sh.at[wid])
    plsc.subcore_barrier()
    sbase = wid * SLICE

    def accum_zero(k, _):
        acc[pl.ds(k * 16, 16)] = zf
        return 0

    lax.fori_loop(0, SLICE // 16, accum_zero, 0)

    def accum_tile(t, _):
        pltpu.sync_copy(part_sh.at[t, pl.ds(sbase, SLICE)],
                        pbuf.at[pl.ds(0, SLICE)])

        def add_v(k, _):
            acc[pl.ds(k * 16, 16)] = (
                acc[pl.ds(k * 16, 16)] + pbuf[pl.ds(k * 16, 16)])
            return 0

        lax.fori_loop(0, SLICE // 16, add_v, 0)
        return 0

    lax.fori_loop(0, NT, accum_tile, 0)

    def dinv_v(k, _):
        d = acc[pl.ds(k * 16, 16)] + 1.0
        acc[pl.ds(k * 16, 16)] = _newton_rsqrt(d)
        return 0

    lax.fori_loop(0, SLICE // 16, dinv_v, 0)
    pltpu.sync_copy(acc.at[pl.ds(0, SLICE)], degdinv_sh.at[pl.ds(sbase, SLICE)])
    plsc.subcore_barrier()
    pltpu.sync_copy(degdinv_sh, dinvl)

    @pl.when(wid == 0)
    def _dedup():
        mone = jnp.full((16,), -1, i32)

        def zero_flag(k, _):
            flagl[pl.ds(k * 16, 16)] = mone
            return 0

        lax.fori_loop(0, NPAD // 16, zero_flag, 0)

        def load_cnts(t, _):
            pltpu.sync_copy(cnts_sh.at[t], cntall.at[pl.ds(t * 16, 16)])
            return 0

        lax.fori_loop(0, NT, load_cnts, 0)

        def dedup_one(s, K, extra):
            f = _sload(flagl, s)
            isnew = f < 0
            slot = jnp.where(isnew, K, f)
            _sstore(flagl, s, slot)
            _sstore(Ul, slot, s)
            dv = _sload(dinvl, s) + extra
            old = jnp.where(isnew, 0.0, _sload(wvl, slot))
            _sstore(wvl, slot, old + dv)
            return K + jnp.where(isnew, 1, 0)

        def seg(t, K):
            cnt = _sload(cntall, t * 16)

            def from_c(K):
                def load(ch, _):
                    pltpu.sync_copy(C_sh.at[t, pl.ds(ch * 256, 256)],
                                    matchb.at[pl.ds(ch * 256, 256)])
                    return 0

                lax.fori_loop(0, (cnt + 255) // 256, load, 0)

                def ent(i, K):
                    return dedup_one(_sload(matchb, i), K, jnp.float32(0.0))

                return lax.fori_loop(0, cnt, ent, K)

            def rescan(K):
                def rw(w, K):
                    tb = t * EPT + w * WIN
                    pltpu.sync_copy(dst_hbm.at[pl.ds(tb, WIN)],
                                    dstw.at[pl.ds(0, WIN)])
                    pltpu.sync_copy(src_hbm.at[pl.ds(tb, WIN)],
                                    srcw.at[pl.ds(0, WIN)])

                    def rv(v, K):
                        off = v * 16
                        dv = dstw[pl.ds(off, 16)]
                        nhit = jnp.sum(jnp.where(dv == idx, 1, 0))

                        def slow(K):
                            def lane(l, K):
                                d = _sload(dstw, off + l)

                                def hitfn(K):
                                    return dedup_one(
                                        _sload(srcw, off + l), K,
                                        jnp.float32(0.0))

                                return lax.cond(d == idx, hitfn,
                                                lambda K: K, K)

                            return lax.fori_loop(0, 16, lane, K)

                        return lax.cond(nhit > 0, slow, lambda K: K, K)

                    return lax.fori_loop(0, WIN // 16, rv, K)

                return lax.fori_loop(0, NWIN, rw, K)

            return lax.cond(cnt <= CSEG, from_c, rescan, K)

        K = lax.fori_loop(0, NT, seg, jnp.int32(0))
        K = dedup_one(idx, K, jnp.float32(0.0))
        kidxl[...] = jnp.broadcast_to(K, (16,)).astype(i32)
        pltpu.sync_copy(kidxl, kidx_sh)
        pltpu.sync_copy(flagl, flag_sh)
        pltpu.sync_copy(Ul, U_sh)
        pltpu.sync_copy(wvl, wv_hbm)

    plsc.subcore_barrier()
    pltpu.sync_copy(flag_sh, flagl)
    pltpu.sync_copy(U_sh, Ul)
    pltpu.sync_copy(kidx_sh, kidxl)
    K = _sload(kidxl, 0)

    def run_batch(ivec, svec, scl):
        idxgbuf[...] = ivec
        idxsbuf[...] = svec
        sclbuf[pl.ds(0, 16)] = scl
        pltpu.async_copy(x_hbm.at[idxgbuf], xrows, sem).wait()

        def scale_row(r, _):
            s = _sload(sclbuf, r)
            sv = jnp.broadcast_to(s, (16,))
            for c in range(D_IN // 16):
                xrows[r, pl.ds(c * 16, 16)] = xrows[r, pl.ds(c * 16, 16)] * sv
            return 0

        lax.fori_loop(0, GB, scale_row, 0)
        pltpu.sync_copy(xrows, a_sh.at[idxsbuf], add=True)

    npass = (K + CAPR - 1) // CAPR

    def one_pass(p, _):
        lo = p * CAPR
        nloc = jnp.minimum(CAPR, K - lo)
        nblk = (nloc + GB - 1) // GB
        nmine = jnp.maximum(0, (nblk - wid + NT - 1) // NT)

        def rezero_xrows(r, _):
            for c in range(D_IN // 16):
                xrows[r, pl.ds(c * 16, 16)] = zf
            return 0

        lax.fori_loop(0, GB, rezero_xrows, 0)

        def zero_blk(ii, _):
            b = wid + ii * NT
            pltpu.sync_copy(xrows, a_sh.at[pl.ds(b * GB, GB)])
            return 0

        lax.fori_loop(0, nmine, zero_blk, 0)
        plsc.subcore_barrier()

        def scan_d_win(w, cnt):
            pltpu.sync_copy(dst_hbm.at[pl.ds(base_e + w * WIN, WIN)],
                            dstw.at[pl.ds(0, WIN)])
            pltpu.sync_copy(src_hbm.at[pl.ds(base_e + w * WIN, WIN)],
                            srcw.at[pl.ds(0, WIN)])

            def scan_d_vec(v, cnt):
                off = v * 16
                dv = dstw[pl.ds(off, 16)]
                fv = plsc.load_gather(flagl, [dv])
                hit = (fv >= lo) & (fv < lo + nloc)
                nhit = jnp.sum(jnp.where(hit, 1, 0))

                def slow(cnt):
                    def lane(l, cnt):
                        q = _sload(flagl, _sload(dstw, off + l))
                        _sstore(srcm, cnt, _sload(srcw, off + l))
                        _sstore(pm, cnt, q - lo)
                        take = (q >= lo) & (q < lo + nloc)
                        return cnt + jnp.where(take, 1, 0)

                    return lax.fori_loop(0, 16, lane, cnt)

                return lax.cond(nhit > 0, slow, lambda c: c, cnt)

            cnt = lax.fori_loop(0, WIN // 16, scan_d_vec, cnt)
            nfull = cnt // GB

            def batch(b, _):
                sv = srcm[pl.ds(b * GB, GB)]
                pv = pm[pl.ds(b * GB, GB)]
                dg = plsc.load_gather(dinvl, [sv])
                run_batch(sv, pv, dg)
                return 0

            lax.fori_loop(0, nfull, batch, 0)
            srcm[pl.ds(0, 16)] = srcm[pl.ds(nfull * GB, 16)]
            pm[pl.ds(0, 16)] = pm[pl.ds(nfull * GB, 16)]
            return cnt - nfull * GB

        cnt_end = lax.fori_loop(0, NWIN, scan_d_win, jnp.int32(0))

        @pl.when(cnt_end > 0)
        def _final_batch():
            valid = lanes < cnt_end
            sv = jnp.where(valid, srcm[pl.ds(0, 16)], 0)
            pv = jnp.where(valid, pm[pl.ds(0, 16)], DUMMY)
            dg = plsc.load_gather(dinvl, [sv])
            scl = jnp.where(valid, dg, 0.0)
            run_batch(sv, pv, scl)

        def self_blk(ii, _):
            b = wid + ii * NT
            uv = Ul[pl.ds(lo + b * GB, GB)]
            valid = b * GB + lanes < nloc
            ivec = jnp.where(valid, uv, 0)
            svec = jnp.where(valid, b * GB + lanes, DUMMY)
            dg = plsc.load_gather(dinvl, [ivec])
            scl = jnp.where(valid, dg, 0.0)
            run_batch(ivec, svec, scl)
            return 0

        lax.fori_loop(0, nmine, self_blk, 0)
        plsc.subcore_barrier()

        def out_blk(ii, _):
            b = wid + ii * NT
            pltpu.sync_copy(a_sh.at[pl.ds(b * GB, GB)],
                            A_hbm.at[pl.ds(lo + b * GB, GB)])
            return 0

        lax.fori_loop(0, nmine, out_blk, 0)
        plsc.subcore_barrier()
        return 0

    lax.fori_loop(0, npass, one_pass, 0)

    def scl_v(k, _):
        off = sbase + k * 16
        uv = Ul[pl.ds(off, 16)]
        valid = off + lanes < K
        uv = jnp.where(valid, uv, 0)
        dg = plsc.load_gather(dinvl, [uv])
        acc[pl.ds(k * 16, 16)] = jnp.where(valid, dg, 0.0)
        return 0

    lax.fori_loop(0, SLICE // 16, scl_v, 0)
    pltpu.sync_copy(acc.at[pl.ds(0, SLICE)], scl_hbm.at[pl.ds(sbase, SLICE)])

    @pl.when(wid == 0)
    def _meta():
        dii = _sload(dinvl, idx)
        kf = K.astype(f32)
        mv = jnp.where(lanes == 0, kf, jnp.where(lanes == 1, dii, 0.0))
        sclbuf[pl.ds(0, 16)] = mv
        pltpu.sync_copy(sclbuf.at[pl.ds(0, 16)], meta_hbm)


@jax.jit
def _sc_call(x, srcv, dstv, ntb):
    mesh = plsc.VectorSubcoreMesh(
        core_axis_name="c", subcore_axis_name="s", num_cores=1)
    f32 = jnp.float32
    i32 = jnp.int32
    kern = pl.kernel(
        _sc_body,
        out_type=[
            jax.ShapeDtypeStruct((NPAD, D_IN), f32),
            jax.ShapeDtypeStruct((NPAD,), f32),
            jax.ShapeDtypeStruct((NPAD,), f32),
            jax.ShapeDtypeStruct((16,), f32),
        ],
        mesh=mesh,
        compiler_params=pltpu.CompilerParams(needs_layout_passes=False),
        scratch_types=[
            pltpu.VMEM((WIN + 16,), i32),
            pltpu.VMEM((WIN + 16,), i32),
            pltpu.VMEM((NPAD,), f32),
            pltpu.VMEM((NPAD,), i32),
            pltpu.VMEM((NPAD,), i32),
            pltpu.VMEM((NPAD,), f32),
            pltpu.VMEM((CSEG + 48,), i32),
            pltpu.VMEM((CCAP,), i32),
            pltpu.VMEM((CCAP,), i32),
            pltpu.VMEM((GB, D_IN), f32),
            pltpu.VMEM((SLICE + 16,), f32),
            pltpu.VMEM((SLICE + 16,), f32),
            pltpu.VMEM((16,), i32),
            pltpu.VMEM((NT * 16 + 16,), i32),
            pltpu.VMEM((16,), i32),
            pltpu.VMEM((GB,), i32),
            pltpu.VMEM((GB,), i32),
            pltpu.VMEM((32,), f32),
            pltpu.VMEM((16,), i32),
            pltpu.VMEM_SHARED((NT, NPAD), f32),
            pltpu.VMEM_SHARED((NPAD,), f32),
            pltpu.VMEM_SHARED((NT, CSEG), i32),
            pltpu.VMEM_SHARED((NT, 16), i32),
            pltpu.VMEM_SHARED((NPAD,), i32),
            pltpu.VMEM_SHARED((NPAD,), i32),
            pltpu.VMEM_SHARED((16,), i32),
            pltpu.VMEM_SHARED((CAPR + 16, D_IN), f32),
            pltpu.SemaphoreType.DMA,
        ],
    )
    return kern(x, srcv, dstv, ntb)


def _tc_body(A_ref, wv_ref, scl_ref, meta_ref,
             W1_ref, b1_ref, W2_ref, b2_ref,
             Wc1_ref, bc1_ref, Wc2_ref, bc2_ref,
             out_ref, u_scr):
    i = pl.program_id(0)
    f32 = jnp.float32
    Ki = meta_ref[0, 0].astype(jnp.int32)

    blk = A_ref[...]
    M = jnp.dot(blk, W1_ref[...], preferred_element_type=f32)
    h1 = jnp.maximum(b1_ref[...] + scl_ref[...] * M, 0.0)
    rid = i * TCB + lax.broadcasted_iota(jnp.int32, (TCB, 1), 0)
    mask = rid < Ki
    part = jnp.sum(jnp.where(mask, wv_ref[...] * h1, 0.0),
                   axis=0, keepdims=True)

    @pl.when(i == 0)
    def _init():
        u_scr[0:1, :] = part

    @pl.when(i > 0)
    def _acc():
        u_scr[0:1, :] = u_scr[0:1, :] + part

    @pl.when(i == NBLK - 1)
    def _tail():
        dii = meta_ref[0, 1]
        u = u_scr[0:1, :]
        h2 = b2_ref[...] + dii * jnp.dot(u, W2_ref[...],
                                         preferred_element_type=f32)
        z = jnp.maximum(jnp.dot(h2, Wc1_ref[...],
                                preferred_element_type=f32) + bc1_ref[...],
                        0.0)
        o = jnp.dot(z, Wc2_ref[...], preferred_element_type=f32) + bc2_ref[...]
        res = 1.0 / (1.0 + jnp.exp(-o))
        out_ref[...] = jnp.broadcast_to(res, (8, 128))


@jax.jit
def _tc_call(A, wv2, scl2, meta2, W1, b12, W2, b22, Wc1, bc12, Wc2p, bc2p):
    f32 = jnp.float32
    return pl.pallas_call(
        _tc_body,
        grid=(NBLK,),
        in_specs=[
            pl.BlockSpec((TCB, D_IN), lambda i: (i, 0)),
            pl.BlockSpec((TCB, 1), lambda i: (i, 0)),
            pl.BlockSpec((TCB, 1), lambda i: (i, 0)),
            pl.BlockSpec((1, 16), lambda i: (0, 0)),
            pl.BlockSpec((D_IN, D_H), lambda i: (0, 0)),
            pl.BlockSpec((1, D_H), lambda i: (0, 0)),
            pl.BlockSpec((D_H, D_H), lambda i: (0, 0)),
            pl.BlockSpec((1, D_H), lambda i: (0, 0)),
            pl.BlockSpec((D_H, D_H), lambda i: (0, 0)),
            pl.BlockSpec((1, D_H), lambda i: (0, 0)),
            pl.BlockSpec((D_H, 128), lambda i: (0, 0)),
            pl.BlockSpec((1, 128), lambda i: (0, 0)),
        ],
        out_specs=pl.BlockSpec((8, 128), lambda i: (0, 0)),
        out_shape=jax.ShapeDtypeStruct((8, 128), f32),
        scratch_shapes=[pltpu.VMEM((8, D_H), f32)],
    )(A, wv2, scl2, meta2, W1, b12, W2, b22, Wc1, bc12, Wc2p, bc2p)


def kernel(x, edge_index, num_timeline, W1, b1, W2, b2, Wc1, bc1, Wc2, bc2):
    srcv = edge_index[0]
    dstv = edge_index[1]
    ntb = jnp.broadcast_to(num_timeline, (16,)).astype(jnp.int32)
    A, wv, scl, meta = _sc_call(x, srcv, dstv, ntb)
    Wc2p = jnp.zeros((D_H, 128), Wc2.dtype).at[:, :D_OUT].set(Wc2)
    bc2p = jnp.zeros((1, 128), bc2.dtype).at[0, :D_OUT].set(bc2)
    out = _tc_call(A, wv[:, None], scl[:, None], meta[None, :],
                   W1, b1[None, :], W2, b2[None, :],
                   Wc1, bc1[None, :], Wc2p, bc2p)
    return out[0:1, :D_OUT]
